# baseline (speedup 1.0000x reference)
import functools

import jax
import jax.numpy as jnp
from jax import lax
from jax.experimental import pallas as pl
from jax.experimental.pallas import tpu as pltpu
from jax.experimental.pallas import tpu_sc as plsc

N_NODES = 10000
N_EDGES = 320000
NUM_HIDDEN = 128
NUM_E = 16
NUM_HEADS = 4
HEAD_DIM = NUM_HIDDEN // NUM_HEADS
ATTN_SCALE = HEAD_DIM ** (-0.5)

NC = 2
NS = 16
NW = NC * NS
C = 128
NCH = 80
EW = NCH * C
EP = NW * EW
NP = 10240
RPT = NP // NS

_sc_mesh = plsc.VectorSubcoreMesh(core_axis_name="c", subcore_axis_name="s")


def _sc_gather_body(src_hbm, dst_hbm, ta_hbm, tc_hbm, ga_hbm, gc_hbm,
                    src_v, dst_v, buf_a, buf_b, sem_a, sem_b):
    cid = lax.axis_index("c")
    sid = lax.axis_index("s")
    wid = sid * NC + cid
    base = wid * EW
    pltpu.sync_copy(src_hbm.at[wid], src_v)
    pltpu.sync_copy(dst_hbm.at[wid], dst_v)

    def run_table(idx_v, tab, out):
        def body(g, carry):
            cps = [pltpu.async_copy(tab.at[idx_v.at[g * 2 + h]],
                                    buf_a.at[pl.ds(h * C, C)], sem_a)
                   for h in range(2)]
            for cp in cps:
                cp.wait()
            pltpu.sync_copy(buf_a, out.at[pl.ds(base + g * 2 * C, 2 * C)])
            return carry

        lax.fori_loop(0, 40, body, 0)

    run_table(src_v, ta_hbm, ga_hbm)
    run_table(dst_v, tc_hbm, gc_hbm)


_sc_gather = functools.partial(
    pl.kernel,
    out_type=(jax.ShapeDtypeStruct((EP, NUM_HIDDEN // 2), jnp.float32),
              jax.ShapeDtypeStruct((EP, NUM_HIDDEN // 2), jnp.float32)),
    mesh=_sc_mesh,
    compiler_params=pltpu.CompilerParams(use_tc_tiling_on_sc=False),
    scratch_types=[
        pltpu.VMEM((NCH, C), jnp.int32),
        pltpu.VMEM((NCH, C), jnp.int32),
        pltpu.VMEM((2 * C, NUM_HIDDEN // 2), jnp.float32),
        pltpu.VMEM((2 * C, NUM_HIDDEN // 2), jnp.float32),
        pltpu.SemaphoreType.DMA,
        pltpu.SemaphoreType.DMA,
    ],
)(_sc_gather_body)


def _sc_scatter_u_body(idx_hbm, u_hbm, z128_hbm, hv_out,
                       idx_v, buf_a, buf_b, hv_sh, sem_a, sem_b):
    cid = lax.axis_index("c")
    sid = lax.axis_index("s")
    wid = sid * NC + cid
    base = wid * EW
    astart = sid * RPT
    pltpu.sync_copy(z128_hbm, hv_sh.at[pl.ds(astart, RPT)])
    pltpu.sync_copy(idx_hbm.at[wid], idx_v)
    plsc.subcore_barrier()

    def body(k, carry):
        pltpu.async_copy(u_hbm.at[pl.ds(base + k * 2 * C, C)],
                         buf_a, sem_a).wait()
        pltpu.sync_copy(buf_a, hv_sh.at[idx_v.at[2 * k]], add=True)
        pltpu.async_copy(u_hbm.at[pl.ds(base + (2 * k + 1) * C, C)],
                         buf_b, sem_b).wait()
        pltpu.sync_copy(buf_b, hv_sh.at[idx_v.at[2 * k + 1]], add=True)
        return carry

    lax.fori_loop(0, NCH // 2, body, 0)
    plsc.subcore_barrier()
    pltpu.sync_copy(hv_sh.at[pl.ds(astart, RPT)],
                    hv_out.at[cid, pl.ds(astart, RPT)])


_sc_scatter_u = functools.partial(
    pl.kernel,
    out_type=jax.ShapeDtypeStruct((NC, NP, NUM_HIDDEN), jnp.float32),
    mesh=_sc_mesh,
    scratch_types=[
        pltpu.VMEM((NCH, C), jnp.int32),
        pltpu.VMEM((C, NUM_HIDDEN), jnp.float32),
        pltpu.VMEM((C, NUM_HIDDEN), jnp.float32),
        pltpu.VMEM_SHARED((NP, NUM_HIDDEN), jnp.float32),
        pltpu.SemaphoreType.DMA,
        pltpu.SemaphoreType.DMA,
    ],
)(_sc_scatter_u_body)


_BN = 1024


def _pack_bf16(p):
    half = NUM_HIDDEN // 2
    lo = p[:, :half].astype(jnp.bfloat16).astype(jnp.float32)
    hi = p[:, half:].astype(jnp.bfloat16).astype(jnp.float32)
    lo_i = jax.lax.bitcast_convert_type(lo, jnp.int32)
    hi_i = jax.lax.bitcast_convert_type(hi, jnp.int32)
    packed = jax.lax.shift_right_logical(lo_i, 16) | hi_i
    return jax.lax.bitcast_convert_type(packed, jnp.float32)


def _unpack_bf16(w):
    w_i = jax.lax.bitcast_convert_type(w, jnp.int32)
    lo = jax.lax.bitcast_convert_type(
        jax.lax.shift_left(w_i, 16), jnp.float32)
    hi = jax.lax.bitcast_convert_type(
        w_i & jnp.int32(-65536), jnp.float32)
    return jnp.concatenate([lo, hi], axis=1)


def _nodeproj_body(hv_ref, b1a_ref, b1c_ref, pa_ref, pc_ref):
    x = hv_ref[...]
    pa_ref[...] = _pack_bf16(
        jnp.dot(x, b1a_ref[...], preferred_element_type=jnp.float32))
    pc_ref[...] = _pack_bf16(
        jnp.dot(x, b1c_ref[...], preferred_element_type=jnp.float32))


def _node_proj(h_Vp, b1a, b1c):
    nb = NP // _BN
    return pl.pallas_call(
        _nodeproj_body,
        grid=(nb,),
        in_specs=[
            pl.BlockSpec((_BN, NUM_HIDDEN), lambda i: (i, 0)),
            pl.BlockSpec((NUM_HIDDEN, NUM_HIDDEN), lambda i: (0, 0)),
            pl.BlockSpec((NUM_HIDDEN, NUM_HIDDEN), lambda i: (0, 0)),
        ],
        out_specs=[
            pl.BlockSpec((_BN, NUM_HIDDEN // 2), lambda i: (i, 0)),
            pl.BlockSpec((_BN, NUM_HIDDEN // 2), lambda i: (i, 0)),
        ],
        out_shape=[
            jax.ShapeDtypeStruct((NP, NUM_HIDDEN // 2), jnp.float32),
            jax.ShapeDtypeStruct((NP, NUM_HIDDEN // 2), jnp.float32),
        ],
        compiler_params=pltpu.CompilerParams(
            dimension_semantics=("parallel",)),
    )(h_Vp, b1a, b1c)


_BE = 2048


def _edge_body(ga_ref, gc_ref, he_ref, b1e_ref, b1b_ref, b2_ref, b2b_ref,
               b3_ref, b3b_ref, wv_ref, wvb_ref, r_ref, ex_ref, u_ref):
    he = he_ref[...]
    w1 = jnp.maximum(
        _unpack_bf16(ga_ref[...]) + _unpack_bf16(gc_ref[...])
        + jnp.dot(he, b1e_ref[...], preferred_element_type=jnp.float32)
        + b1b_ref[...], 0.0)
    w2 = jnp.maximum(
        jnp.dot(w1, b2_ref[...], preferred_element_type=jnp.float32)
        + b2b_ref[...], 0.0)
    lg = (jnp.dot(w2, b3_ref[...], preferred_element_type=jnp.float32)
          + b3b_ref[...]) * ATTN_SCALE
    ex4 = jnp.exp(lg[:, :NUM_HEADS])
    x = (jnp.dot(he, wv_ref[...], preferred_element_type=jnp.float32)
         + wvb_ref[...])
    v = x * 0.5 * (1.0 + lax.erf(x * (2.0 ** -0.5)))
    exb = jnp.dot(ex4, r_ref[...], preferred_element_type=jnp.float32)
    ex_ref[...] = exb
    u_ref[...] = v * exb


def _edge_stage(ga, gc, h_Ep, b1e, b1b, B2_w, b2b, b3p, b3bp, W_V_w, wvb, R):
    nb = EP // _BE
    full = lambda i: (0, 0)
    return pl.pallas_call(
        _edge_body,
        grid=(nb,),
        in_specs=[
            pl.BlockSpec((_BE, NUM_HIDDEN // 2), lambda i: (i, 0)),
            pl.BlockSpec((_BE, NUM_HIDDEN // 2), lambda i: (i, 0)),
            pl.BlockSpec((_BE, NUM_E), lambda i: (i, 0)),
            pl.BlockSpec((NUM_E, NUM_HIDDEN), full),
            pl.BlockSpec((1, NUM_HIDDEN), full),
            pl.BlockSpec((NUM_HIDDEN, NUM_HIDDEN), full),
            pl.BlockSpec((1, NUM_HIDDEN), full),
            pl.BlockSpec((NUM_HIDDEN, NUM_HIDDEN), full),
            pl.BlockSpec((1, NUM_HIDDEN), full),
            pl.BlockSpec((NUM_E, NUM_HIDDEN), full),
            pl.BlockSpec((1, NUM_HIDDEN), full),
            pl.BlockSpec((NUM_HEADS, NUM_HIDDEN), full),
        ],
        out_specs=[
            pl.BlockSpec((_BE, NUM_HIDDEN), lambda i: (i, 0)),
            pl.BlockSpec((_BE, NUM_HIDDEN), lambda i: (i, 0)),
        ],
        out_shape=[
            jax.ShapeDtypeStruct((EP, NUM_HIDDEN), jnp.float32),
            jax.ShapeDtypeStruct((EP, NUM_HIDDEN), jnp.float32),
        ],
        compiler_params=pltpu.CompilerParams(
            dimension_semantics=("parallel",)),
    )(ga, gc, h_Ep, b1e, b1b, B2_w, b2b, b3p, b3bp, W_V_w, wvb, R)


_BO = 1000


def _out_body(n0_ref, n1_ref, s0_ref, s1_ref, hv_ref, wo_ref, gw_ref, gb_ref,
              out_ref):
    num = n0_ref[0] + n1_ref[0]
    den = s0_ref[0] + s1_ref[0] + 1e-16
    hv = num / den
    gate = jax.nn.sigmoid(
        jnp.dot(hv, gw_ref[...], preferred_element_type=jnp.float32)
        + gb_ref[...])
    out_ref[...] = hv_ref[...] + jnp.dot(
        hv, wo_ref[...], preferred_element_type=jnp.float32) * gate


def _node_out(hvn, ssn, h_V, W_O_w, gate_w, gb):
    nb = N_NODES // _BO
    full = lambda i: (0, 0)
    return pl.pallas_call(
        _out_body,
        grid=(nb,),
        in_specs=[
            pl.BlockSpec((1, _BO, NUM_HIDDEN), lambda i: (0, i, 0)),
            pl.BlockSpec((1, _BO, NUM_HIDDEN), lambda i: (1, i, 0)),
            pl.BlockSpec((1, _BO, NUM_HIDDEN), lambda i: (0, i, 0)),
            pl.BlockSpec((1, _BO, NUM_HIDDEN), lambda i: (1, i, 0)),
            pl.BlockSpec((_BO, NUM_HIDDEN), lambda i: (i, 0)),
            pl.BlockSpec((NUM_HIDDEN, NUM_HIDDEN), full),
            pl.BlockSpec((NUM_HIDDEN, NUM_HIDDEN), full),
            pl.BlockSpec((1, NUM_HIDDEN), full),
        ],
        out_specs=pl.BlockSpec((_BO, NUM_HIDDEN), lambda i: (i, 0)),
        out_shape=jax.ShapeDtypeStruct((N_NODES, NUM_HIDDEN), jnp.float32),
        compiler_params=pltpu.CompilerParams(
            dimension_semantics=("parallel",)),
    )(hvn, hvn, ssn, ssn, h_V, W_O_w, gate_w, gb)


def kernel(h_V, h_E, edge_idx, W_V_w, W_V_b, B1_w, B1_b, B2_w, B2_b,
           B3_w, B3_b, W_O_w, gate_w, gate_b):
    f32 = jnp.float32
    npad = EP - N_EDGES
    trash = (N_NODES + jnp.arange(npad, dtype=jnp.int32) % (NP - N_NODES))
    src = jnp.concatenate([edge_idx[0].astype(jnp.int32), trash]
                          ).reshape(NW, NCH, C)
    dst = jnp.concatenate([edge_idx[1].astype(jnp.int32), trash]
                          ).reshape(NW, NCH, C)
    h_Vp = jnp.pad(h_V, ((0, NP - N_NODES), (0, 0)))
    h_Ep = jnp.pad(h_E, ((0, npad), (0, 0)))

    b1a = B1_w[:NUM_HIDDEN]
    b1e = B1_w[NUM_HIDDEN:NUM_HIDDEN + NUM_E]
    b1c = B1_w[NUM_HIDDEN + NUM_E:]
    b1b = B1_b.reshape(1, NUM_HIDDEN)
    b2b = B2_b.reshape(1, NUM_HIDDEN)
    b3p = jnp.zeros((NUM_HIDDEN, NUM_HIDDEN), f32).at[:, :NUM_HEADS].set(B3_w)
    b3bp = jnp.zeros((1, NUM_HIDDEN), f32).at[0, :NUM_HEADS].set(B3_b)
    wvb = W_V_b.reshape(1, NUM_HIDDEN)
    gb = gate_b.reshape(1, NUM_HIDDEN)
    cols = jnp.arange(NUM_HIDDEN, dtype=jnp.int32) // HEAD_DIM
    R = (cols[None, :] == jnp.arange(NUM_HEADS, dtype=jnp.int32)[:, None]
         ).astype(f32)
    z128 = jnp.zeros((RPT, NUM_HIDDEN), f32)

    pa, pc = _node_proj(h_Vp, b1a, b1c)
    ga, gc = _sc_gather(src, dst, pa, pc)
    exb, u = _edge_stage(ga, gc, h_Ep, b1e, b1b, B2_w, b2b, b3p, b3bp,
                         W_V_w, wvb, R)
    hvn = _sc_scatter_u(src, u, z128)
    ssn = _sc_scatter_u(src, exb, z128)
    return _node_out(hvn, ssn, h_V, W_O_w, gate_w, gb)


# speedup vs baseline: 1.1175x; 1.1175x over previous
import functools

import jax
import jax.numpy as jnp
from jax import lax
from jax.experimental import pallas as pl
from jax.experimental.pallas import tpu as pltpu
from jax.experimental.pallas import tpu_sc as plsc

N_NODES = 10000
N_EDGES = 320000
NUM_HIDDEN = 128
NUM_E = 16
NUM_HEADS = 4
HEAD_DIM = NUM_HIDDEN // NUM_HEADS
ATTN_SCALE = HEAD_DIM ** (-0.5)

NC = 2
NS = 16
NW = NC * NS
C = 128
NCH = 80
EW = NCH * C
EP = NW * EW
NP = 10240
RPT = NP // NS

EPF = EP * NUM_HEADS // 128
EWF = EPF // NW
NPF = NP * NUM_HEADS // 128
NRED = 10

_sc_mesh = plsc.VectorSubcoreMesh(core_axis_name="c", subcore_axis_name="s")


def _sc_gather_body(src_hbm, dst_hbm, ta_hbm, tc_hbm, ga_hbm, gc_hbm,
                    src_v, dst_v, buf_a, sem_a):
    cid = lax.axis_index("c")
    sid = lax.axis_index("s")
    wid = sid * NC + cid
    base = wid * EW
    pltpu.sync_copy(src_hbm.at[wid], src_v)
    pltpu.sync_copy(dst_hbm.at[wid], dst_v)

    def run_table(idx_v, tab, out):
        def body(g, carry):
            cps = [pltpu.async_copy(tab.at[idx_v.at[g * 2 + h]],
                                    buf_a.at[pl.ds(h * C, C)], sem_a)
                   for h in range(2)]
            for cp in cps:
                cp.wait()
            pltpu.sync_copy(buf_a, out.at[pl.ds(base + g * 2 * C, 2 * C)])
            return carry

        lax.fori_loop(0, 40, body, 0)

    run_table(src_v, ta_hbm, ga_hbm)
    run_table(dst_v, tc_hbm, gc_hbm)


_sc_gather = functools.partial(
    pl.kernel,
    out_type=(jax.ShapeDtypeStruct((EP, NUM_HIDDEN), jnp.float32),
              jax.ShapeDtypeStruct((EP, NUM_HIDDEN), jnp.float32)),
    mesh=_sc_mesh,
    scratch_types=[
        pltpu.VMEM((NCH, C), jnp.int32),
        pltpu.VMEM((NCH, C), jnp.int32),
        pltpu.VMEM((2 * C, NUM_HIDDEN), jnp.float32),
        pltpu.SemaphoreType.DMA,
    ],
)(_sc_gather_body)


def _sc_scatter_u_body(idx_hbm, u_hbm, z128_hbm, hv_out,
                       idx_v, buf_a, buf_b, hv_sh, sem_a, sem_b):
    cid = lax.axis_index("c")
    sid = lax.axis_index("s")
    wid = sid * NC + cid
    base = wid * EW
    astart = sid * RPT
    pltpu.sync_copy(z128_hbm, hv_sh.at[pl.ds(astart, RPT)])
    pltpu.sync_copy(idx_hbm.at[wid], idx_v)
    plsc.subcore_barrier()

    def body(k, carry):
        pltpu.async_copy(u_hbm.at[pl.ds(base + k * 2 * C, C)],
                         buf_a, sem_a).wait()
        pltpu.sync_copy(buf_a, hv_sh.at[idx_v.at[2 * k]], add=True)
        pltpu.async_copy(u_hbm.at[pl.ds(base + (2 * k + 1) * C, C)],
                         buf_b, sem_b).wait()
        pltpu.sync_copy(buf_b, hv_sh.at[idx_v.at[2 * k + 1]], add=True)
        return carry

    lax.fori_loop(0, NCH // 2, body, 0)
    plsc.subcore_barrier()
    pltpu.sync_copy(hv_sh.at[pl.ds(astart, RPT)],
                    hv_out.at[cid, pl.ds(astart, RPT)])


_sc_scatter_u = functools.partial(
    pl.kernel,
    out_type=jax.ShapeDtypeStruct((NC, NP, NUM_HIDDEN), jnp.float32),
    mesh=_sc_mesh,
    scratch_types=[
        pltpu.VMEM((NCH, C), jnp.int32),
        pltpu.VMEM((C, NUM_HIDDEN), jnp.float32),
        pltpu.VMEM((C, NUM_HIDDEN), jnp.float32),
        pltpu.VMEM_SHARED((NP, NUM_HIDDEN), jnp.float32),
        pltpu.SemaphoreType.DMA,
        pltpu.SemaphoreType.DMA,
    ],
)(_sc_scatter_u_body)


def _sc_scatter_e_body(idx_hbm, e_hbm, z16_hbm, ss_out,
                       idx_v, buf_a, buf_b, ss_sh, sem_a, sem_b):
    cid = lax.axis_index("c")
    sid = lax.axis_index("s")
    wid = sid * NC + cid
    base = wid * EW
    astart = sid * RPT
    pltpu.sync_copy(z16_hbm, ss_sh.at[pl.ds(astart, RPT)])
    pltpu.sync_copy(idx_hbm.at[wid], idx_v)
    plsc.subcore_barrier()

    def body(k, carry):
        pltpu.async_copy(e_hbm.at[pl.ds(base + k * 2 * C, C)],
                         buf_a, sem_a).wait()
        pltpu.sync_copy(buf_a, ss_sh.at[idx_v.at[2 * k]], add=True)
        pltpu.async_copy(e_hbm.at[pl.ds(base + (2 * k + 1) * C, C)],
                         buf_b, sem_b).wait()
        pltpu.sync_copy(buf_b, ss_sh.at[idx_v.at[2 * k + 1]], add=True)
        return carry

    lax.fori_loop(0, NCH // 2, body, 0)
    plsc.subcore_barrier()
    pltpu.sync_copy(ss_sh.at[pl.ds(astart, RPT)],
                    ss_out.at[cid, pl.ds(astart, RPT)])


_EX_W = 16


_sc_scatter_e = functools.partial(
    pl.kernel,
    out_type=jax.ShapeDtypeStruct((NC, NP, _EX_W), jnp.float32),
    mesh=_sc_mesh,
    compiler_params=pltpu.CompilerParams(use_tc_tiling_on_sc=False),
    scratch_types=[
        pltpu.VMEM((NCH, C), jnp.int32),
        pltpu.VMEM((C, _EX_W), jnp.float32),
        pltpu.VMEM((C, _EX_W), jnp.float32),
        pltpu.VMEM_SHARED((NP, _EX_W), jnp.float32),
        pltpu.SemaphoreType.DMA,
        pltpu.SemaphoreType.DMA,
    ],
)(_sc_scatter_e_body)


_BN = 1024


def _nodeproj_body(hv_ref, b1a_ref, b1c_ref, pa_ref, pc_ref):
    x = hv_ref[...]
    pa_ref[...] = jnp.dot(x, b1a_ref[...], preferred_element_type=jnp.float32)
    pc_ref[...] = jnp.dot(x, b1c_ref[...], preferred_element_type=jnp.float32)


def _node_proj(h_Vp, b1a, b1c):
    nb = NP // _BN
    return pl.pallas_call(
        _nodeproj_body,
        grid=(nb,),
        in_specs=[
            pl.BlockSpec((_BN, NUM_HIDDEN), lambda i: (i, 0)),
            pl.BlockSpec((NUM_HIDDEN, NUM_HIDDEN), lambda i: (0, 0)),
            pl.BlockSpec((NUM_HIDDEN, NUM_HIDDEN), lambda i: (0, 0)),
        ],
        out_specs=[
            pl.BlockSpec((_BN, NUM_HIDDEN), lambda i: (i, 0)),
            pl.BlockSpec((_BN, NUM_HIDDEN), lambda i: (i, 0)),
        ],
        out_shape=[
            jax.ShapeDtypeStruct((NP, NUM_HIDDEN), jnp.float32),
            jax.ShapeDtypeStruct((NP, NUM_HIDDEN), jnp.float32),
        ],
        compiler_params=pltpu.CompilerParams(
            dimension_semantics=("parallel",)),
    )(h_Vp, b1a, b1c)


_BE = 2048


def _edge_body(ga_ref, gc_ref, he_ref, b1e_ref, b1b_ref, b2_ref, b2b_ref,
               b3_ref, b3b_ref, wv_ref, wvb_ref, r_ref, ex_ref, u_ref):
    he = he_ref[...]
    w1 = jnp.maximum(
        ga_ref[...] + gc_ref[...]
        + jnp.dot(he, b1e_ref[...], preferred_element_type=jnp.float32)
        + b1b_ref[...], 0.0)
    w2 = jnp.maximum(
        jnp.dot(w1, b2_ref[...], preferred_element_type=jnp.float32)
        + b2b_ref[...], 0.0)
    lg = (jnp.dot(w2, b3_ref[...], preferred_element_type=jnp.float32)
          + b3b_ref[...]) * ATTN_SCALE
    ex4 = jnp.exp(lg[:, :NUM_HEADS])
    ex_ref[...] = jnp.concatenate([ex4, ex4, ex4, ex4], axis=1)
    x = (jnp.dot(he, wv_ref[...], preferred_element_type=jnp.float32)
         + wvb_ref[...])
    v = x * 0.5 * (1.0 + lax.erf(x * (2.0 ** -0.5)))
    exb = jnp.dot(ex4, r_ref[...], preferred_element_type=jnp.float32)
    u_ref[...] = v * exb


def _edge_stage(ga, gc, h_Ep, b1e, b1b, B2_w, b2b, b3p, b3bp, W_V_w, wvb, R):
    nb = EP // _BE
    full = lambda i: (0, 0)
    return pl.pallas_call(
        _edge_body,
        grid=(nb,),
        in_specs=[
            pl.BlockSpec((_BE, NUM_HIDDEN), lambda i: (i, 0)),
            pl.BlockSpec((_BE, NUM_HIDDEN), lambda i: (i, 0)),
            pl.BlockSpec((_BE, NUM_E), lambda i: (i, 0)),
            pl.BlockSpec((NUM_E, NUM_HIDDEN), full),
            pl.BlockSpec((1, NUM_HIDDEN), full),
            pl.BlockSpec((NUM_HIDDEN, NUM_HIDDEN), full),
            pl.BlockSpec((1, NUM_HIDDEN), full),
            pl.BlockSpec((NUM_HIDDEN, NUM_HIDDEN), full),
            pl.BlockSpec((1, NUM_HIDDEN), full),
            pl.BlockSpec((NUM_E, NUM_HIDDEN), full),
            pl.BlockSpec((1, NUM_HIDDEN), full),
            pl.BlockSpec((NUM_HEADS, NUM_HIDDEN), full),
        ],
        out_specs=[
            pl.BlockSpec((_BE, 4 * NUM_HEADS), lambda i: (i, 0)),
            pl.BlockSpec((_BE, NUM_HIDDEN), lambda i: (i, 0)),
        ],
        out_shape=[
            jax.ShapeDtypeStruct((EP, 4 * NUM_HEADS), jnp.float32),
            jax.ShapeDtypeStruct((EP, NUM_HIDDEN), jnp.float32),
        ],
        compiler_params=pltpu.CompilerParams(
            dimension_semantics=("parallel",)),
    )(ga, gc, h_Ep, b1e, b1b, B2_w, b2b, b3p, b3bp, W_V_w, wvb, R)


_BO = 1000


def _out_body(n0_ref, n1_ref, s0_ref, s1_ref, hv_ref, wo_ref, gw_ref, gb_ref,
              r_ref, out_ref):
    num = n0_ref[0] + n1_ref[0]
    ss = s0_ref[0] + s1_ref[0]
    den = jnp.dot(ss, r_ref[...], preferred_element_type=jnp.float32) + 1e-16
    hv = num / den
    gate = jax.nn.sigmoid(
        jnp.dot(hv, gw_ref[...], preferred_element_type=jnp.float32)
        + gb_ref[...])
    out_ref[...] = hv_ref[...] + jnp.dot(
        hv, wo_ref[...], preferred_element_type=jnp.float32) * gate


def _node_out(hvn, ssn, h_V, W_O_w, gate_w, gb, R):
    nb = N_NODES // _BO
    full = lambda i: (0, 0)
    return pl.pallas_call(
        _out_body,
        grid=(nb,),
        in_specs=[
            pl.BlockSpec((1, _BO, NUM_HIDDEN), lambda i: (0, i, 0)),
            pl.BlockSpec((1, _BO, NUM_HIDDEN), lambda i: (1, i, 0)),
            pl.BlockSpec((1, _BO, 4 * NUM_HEADS), lambda i: (0, i, 0)),
            pl.BlockSpec((1, _BO, 4 * NUM_HEADS), lambda i: (1, i, 0)),
            pl.BlockSpec((_BO, NUM_HIDDEN), lambda i: (i, 0)),
            pl.BlockSpec((NUM_HIDDEN, NUM_HIDDEN), full),
            pl.BlockSpec((NUM_HIDDEN, NUM_HIDDEN), full),
            pl.BlockSpec((1, NUM_HIDDEN), full),
            pl.BlockSpec((4 * NUM_HEADS, NUM_HIDDEN), full),
        ],
        out_specs=pl.BlockSpec((_BO, NUM_HIDDEN), lambda i: (i, 0)),
        out_shape=jax.ShapeDtypeStruct((N_NODES, NUM_HIDDEN), jnp.float32),
        compiler_params=pltpu.CompilerParams(
            dimension_semantics=("parallel",)),
    )(hvn, hvn, ssn, ssn, h_V, W_O_w, gate_w, gb, R)


def kernel(h_V, h_E, edge_idx, W_V_w, W_V_b, B1_w, B1_b, B2_w, B2_b,
           B3_w, B3_b, W_O_w, gate_w, gate_b):
    f32 = jnp.float32
    npad = EP - N_EDGES
    trash = (N_NODES + jnp.arange(npad, dtype=jnp.int32) % (NP - N_NODES))
    src = jnp.concatenate([edge_idx[0].astype(jnp.int32), trash]
                          ).reshape(NW, NCH, C)
    dst = jnp.concatenate([edge_idx[1].astype(jnp.int32), trash]
                          ).reshape(NW, NCH, C)
    h_Vp = jnp.pad(h_V, ((0, NP - N_NODES), (0, 0)))
    h_Ep = jnp.pad(h_E, ((0, npad), (0, 0)))

    b1a = B1_w[:NUM_HIDDEN]
    b1e = B1_w[NUM_HIDDEN:NUM_HIDDEN + NUM_E]
    b1c = B1_w[NUM_HIDDEN + NUM_E:]
    b1b = B1_b.reshape(1, NUM_HIDDEN)
    b2b = B2_b.reshape(1, NUM_HIDDEN)
    b3p = jnp.zeros((NUM_HIDDEN, NUM_HIDDEN), f32).at[:, :NUM_HEADS].set(B3_w)
    b3bp = jnp.zeros((1, NUM_HIDDEN), f32).at[0, :NUM_HEADS].set(B3_b)
    wvb = W_V_b.reshape(1, NUM_HIDDEN)
    gb = gate_b.reshape(1, NUM_HIDDEN)
    cols = jnp.arange(NUM_HIDDEN, dtype=jnp.int32) // HEAD_DIM
    R = (cols[None, :] == jnp.arange(NUM_HEADS, dtype=jnp.int32)[:, None]
         ).astype(f32)
    M16 = (cols[None, :] == jnp.arange(16, dtype=jnp.int32)[:, None]
           ).astype(f32)
    z128 = jnp.zeros((RPT, NUM_HIDDEN), f32)
    z16 = jnp.zeros((RPT, 16), f32)

    pa, pc = _node_proj(h_Vp, b1a, b1c)
    ga, gc = _sc_gather(src, dst, pa, pc)
    ex, u = _edge_stage(ga, gc, h_Ep, b1e, b1b, B2_w, b2b, b3p, b3bp,
                        W_V_w, wvb, R)
    hvn = _sc_scatter_u(src, u, z128)
    ssn = _sc_scatter_e(src, ex, z16)
    return _node_out(hvn, ssn, h_V, W_O_w, gate_w, gb, M16)


# speedup vs baseline: 1.1530x; 1.0317x over previous
import functools

import jax
import jax.numpy as jnp
from jax import lax
from jax.experimental import pallas as pl
from jax.experimental.pallas import tpu as pltpu
from jax.experimental.pallas import tpu_sc as plsc

N_NODES = 10000
N_EDGES = 320000
NUM_HIDDEN = 128
NUM_E = 16
NUM_HEADS = 4
HEAD_DIM = NUM_HIDDEN // NUM_HEADS
ATTN_SCALE = HEAD_DIM ** (-0.5)

NC = 2
NS = 16
NW = NC * NS
C = 128
NCH = 80
EW = NCH * C
EP = NW * EW
NP = 10240
RPT = NP // NS

EPF = EP * NUM_HEADS // 128
EWF = EPF // NW
NPF = NP * NUM_HEADS // 128
NRED = 10

_sc_mesh = plsc.VectorSubcoreMesh(core_axis_name="c", subcore_axis_name="s")


def _sc_gather_body(src_hbm, dst_hbm, ta_hbm, tc_hbm, ga_hbm, gc_hbm,
                    src_v, dst_v, buf_a, sem_a):
    cid = lax.axis_index("c")
    sid = lax.axis_index("s")
    wid = sid * NC + cid
    base = wid * EW
    pltpu.sync_copy(src_hbm.at[wid], src_v)
    pltpu.sync_copy(dst_hbm.at[wid], dst_v)

    def run_table(idx_v, tab, out):
        def body(g, carry):
            cps = [pltpu.async_copy(tab.at[idx_v.at[g * 4 + h]],
                                    buf_a.at[pl.ds(h * C, C)], sem_a)
                   for h in range(4)]
            for cp in cps:
                cp.wait()
            pltpu.sync_copy(buf_a, out.at[pl.ds(base + g * 4 * C, 4 * C)])
            return carry

        lax.fori_loop(0, 20, body, 0)

    run_table(src_v, ta_hbm, ga_hbm)
    run_table(dst_v, tc_hbm, gc_hbm)


_sc_gather = functools.partial(
    pl.kernel,
    out_type=(jax.ShapeDtypeStruct((EP, NUM_HIDDEN), jnp.float32),
              jax.ShapeDtypeStruct((EP, NUM_HIDDEN), jnp.float32)),
    mesh=_sc_mesh,
    scratch_types=[
        pltpu.VMEM((NCH, C), jnp.int32),
        pltpu.VMEM((NCH, C), jnp.int32),
        pltpu.VMEM((4 * C, NUM_HIDDEN), jnp.float32),
        pltpu.SemaphoreType.DMA,
    ],
)(_sc_gather_body)


def _sc_scatter_u_body(idx_hbm, u_hbm, z128_hbm, hv_out,
                       idx_v, buf_a, buf_b, hv_sh, sem_a, sem_b):
    cid = lax.axis_index("c")
    sid = lax.axis_index("s")
    wid = sid * NC + cid
    base = wid * EW
    astart = sid * RPT
    pltpu.sync_copy(z128_hbm, hv_sh.at[pl.ds(astart, RPT)])
    pltpu.sync_copy(idx_hbm.at[wid], idx_v)
    plsc.subcore_barrier()

    def body(k, carry):
        pltpu.async_copy(u_hbm.at[pl.ds(base + k * 2 * C, C)],
                         buf_a, sem_a).wait()
        pltpu.sync_copy(buf_a, hv_sh.at[idx_v.at[2 * k]], add=True)
        pltpu.async_copy(u_hbm.at[pl.ds(base + (2 * k + 1) * C, C)],
                         buf_b, sem_b).wait()
        pltpu.sync_copy(buf_b, hv_sh.at[idx_v.at[2 * k + 1]], add=True)
        return carry

    lax.fori_loop(0, NCH // 2, body, 0)
    plsc.subcore_barrier()
    pltpu.sync_copy(hv_sh.at[pl.ds(astart, RPT)],
                    hv_out.at[cid, pl.ds(astart, RPT)])


_sc_scatter_u = functools.partial(
    pl.kernel,
    out_type=jax.ShapeDtypeStruct((NC, NP, NUM_HIDDEN), jnp.float32),
    mesh=_sc_mesh,
    scratch_types=[
        pltpu.VMEM((NCH, C), jnp.int32),
        pltpu.VMEM((C, NUM_HIDDEN), jnp.float32),
        pltpu.VMEM((C, NUM_HIDDEN), jnp.float32),
        pltpu.VMEM_SHARED((NP, NUM_HIDDEN), jnp.float32),
        pltpu.SemaphoreType.DMA,
        pltpu.SemaphoreType.DMA,
    ],
)(_sc_scatter_u_body)


_BN = 1024


def _nodeproj_body(hv_ref, b1a_ref, b1c_ref, pa_ref, pc_ref):
    x = hv_ref[...]
    pa_ref[...] = jnp.dot(x, b1a_ref[...], preferred_element_type=jnp.float32)
    pc_ref[...] = jnp.dot(x, b1c_ref[...], preferred_element_type=jnp.float32)


def _node_proj(h_Vp, b1a, b1c):
    nb = NP // _BN
    return pl.pallas_call(
        _nodeproj_body,
        grid=(nb,),
        in_specs=[
            pl.BlockSpec((_BN, NUM_HIDDEN), lambda i: (i, 0)),
            pl.BlockSpec((NUM_HIDDEN, NUM_HIDDEN), lambda i: (0, 0)),
            pl.BlockSpec((NUM_HIDDEN, NUM_HIDDEN), lambda i: (0, 0)),
        ],
        out_specs=[
            pl.BlockSpec((_BN, NUM_HIDDEN), lambda i: (i, 0)),
            pl.BlockSpec((_BN, NUM_HIDDEN), lambda i: (i, 0)),
        ],
        out_shape=[
            jax.ShapeDtypeStruct((NP, NUM_HIDDEN), jnp.float32),
            jax.ShapeDtypeStruct((NP, NUM_HIDDEN), jnp.float32),
        ],
        compiler_params=pltpu.CompilerParams(
            dimension_semantics=("parallel",)),
    )(h_Vp, b1a, b1c)


_BE = 2048


def _edge_body(ga_ref, gc_ref, he_ref, b1e_ref, b1b_ref, b2_ref, b2b_ref,
               b3_ref, b3b_ref, wv_ref, wvb_ref, r_ref, ex_ref, u_ref):
    he = he_ref[...]
    w1 = jnp.maximum(
        ga_ref[...] + gc_ref[...]
        + jnp.dot(he, b1e_ref[...], preferred_element_type=jnp.float32)
        + b1b_ref[...], 0.0)
    w2 = jnp.maximum(
        jnp.dot(w1, b2_ref[...], preferred_element_type=jnp.float32)
        + b2b_ref[...], 0.0)
    lg = (jnp.dot(w2, b3_ref[...], preferred_element_type=jnp.float32)
          + b3b_ref[...]) * ATTN_SCALE
    ex4 = jnp.exp(lg[:, :NUM_HEADS])
    x = (jnp.dot(he, wv_ref[...], preferred_element_type=jnp.float32)
         + wvb_ref[...])
    v = x * 0.5 * (1.0 + lax.erf(x * (2.0 ** -0.5)))
    exb = jnp.dot(ex4, r_ref[...], preferred_element_type=jnp.float32)
    ex_ref[...] = exb
    u_ref[...] = v * exb


def _edge_stage(ga, gc, h_Ep, b1e, b1b, B2_w, b2b, b3p, b3bp, W_V_w, wvb, R):
    nb = EP // _BE
    full = lambda i: (0, 0)
    return pl.pallas_call(
        _edge_body,
        grid=(nb,),
        in_specs=[
            pl.BlockSpec((_BE, NUM_HIDDEN), lambda i: (i, 0)),
            pl.BlockSpec((_BE, NUM_HIDDEN), lambda i: (i, 0)),
            pl.BlockSpec((_BE, NUM_E), lambda i: (i, 0)),
            pl.BlockSpec((NUM_E, NUM_HIDDEN), full),
            pl.BlockSpec((1, NUM_HIDDEN), full),
            pl.BlockSpec((NUM_HIDDEN, NUM_HIDDEN), full),
            pl.BlockSpec((1, NUM_HIDDEN), full),
            pl.BlockSpec((NUM_HIDDEN, NUM_HIDDEN), full),
            pl.BlockSpec((1, NUM_HIDDEN), full),
            pl.BlockSpec((NUM_E, NUM_HIDDEN), full),
            pl.BlockSpec((1, NUM_HIDDEN), full),
            pl.BlockSpec((NUM_HEADS, NUM_HIDDEN), full),
        ],
        out_specs=[
            pl.BlockSpec((_BE, NUM_HIDDEN), lambda i: (i, 0)),
            pl.BlockSpec((_BE, NUM_HIDDEN), lambda i: (i, 0)),
        ],
        out_shape=[
            jax.ShapeDtypeStruct((EP, NUM_HIDDEN), jnp.float32),
            jax.ShapeDtypeStruct((EP, NUM_HIDDEN), jnp.float32),
        ],
        compiler_params=pltpu.CompilerParams(
            dimension_semantics=("parallel",)),
    )(ga, gc, h_Ep, b1e, b1b, B2_w, b2b, b3p, b3bp, W_V_w, wvb, R)


_BO = 1000


def _out_body(n0_ref, n1_ref, s0_ref, s1_ref, hv_ref, wo_ref, gw_ref, gb_ref,
              out_ref):
    num = n0_ref[0] + n1_ref[0]
    den = s0_ref[0] + s1_ref[0] + 1e-16
    hv = num / den
    gate = jax.nn.sigmoid(
        jnp.dot(hv, gw_ref[...], preferred_element_type=jnp.float32)
        + gb_ref[...])
    out_ref[...] = hv_ref[...] + jnp.dot(
        hv, wo_ref[...], preferred_element_type=jnp.float32) * gate


def _node_out(hvn, ssn, h_V, W_O_w, gate_w, gb):
    nb = N_NODES // _BO
    full = lambda i: (0, 0)
    return pl.pallas_call(
        _out_body,
        grid=(nb,),
        in_specs=[
            pl.BlockSpec((1, _BO, NUM_HIDDEN), lambda i: (0, i, 0)),
            pl.BlockSpec((1, _BO, NUM_HIDDEN), lambda i: (1, i, 0)),
            pl.BlockSpec((1, _BO, NUM_HIDDEN), lambda i: (0, i, 0)),
            pl.BlockSpec((1, _BO, NUM_HIDDEN), lambda i: (1, i, 0)),
            pl.BlockSpec((_BO, NUM_HIDDEN), lambda i: (i, 0)),
            pl.BlockSpec((NUM_HIDDEN, NUM_HIDDEN), full),
            pl.BlockSpec((NUM_HIDDEN, NUM_HIDDEN), full),
            pl.BlockSpec((1, NUM_HIDDEN), full),
        ],
        out_specs=pl.BlockSpec((_BO, NUM_HIDDEN), lambda i: (i, 0)),
        out_shape=jax.ShapeDtypeStruct((N_NODES, NUM_HIDDEN), jnp.float32),
        compiler_params=pltpu.CompilerParams(
            dimension_semantics=("parallel",)),
    )(hvn, hvn, ssn, ssn, h_V, W_O_w, gate_w, gb)


def kernel(h_V, h_E, edge_idx, W_V_w, W_V_b, B1_w, B1_b, B2_w, B2_b,
           B3_w, B3_b, W_O_w, gate_w, gate_b):
    f32 = jnp.float32
    npad = EP - N_EDGES
    trash = (N_NODES + jnp.arange(npad, dtype=jnp.int32) % (NP - N_NODES))
    src = jnp.concatenate([edge_idx[0].astype(jnp.int32), trash]
                          ).reshape(NW, NCH, C)
    dst = jnp.concatenate([edge_idx[1].astype(jnp.int32), trash]
                          ).reshape(NW, NCH, C)
    h_Vp = jnp.pad(h_V, ((0, NP - N_NODES), (0, 0)))
    h_Ep = jnp.pad(h_E, ((0, npad), (0, 0)))

    b1a = B1_w[:NUM_HIDDEN]
    b1e = B1_w[NUM_HIDDEN:NUM_HIDDEN + NUM_E]
    b1c = B1_w[NUM_HIDDEN + NUM_E:]
    b1b = B1_b.reshape(1, NUM_HIDDEN)
    b2b = B2_b.reshape(1, NUM_HIDDEN)
    b3p = jnp.zeros((NUM_HIDDEN, NUM_HIDDEN), f32).at[:, :NUM_HEADS].set(B3_w)
    b3bp = jnp.zeros((1, NUM_HIDDEN), f32).at[0, :NUM_HEADS].set(B3_b)
    wvb = W_V_b.reshape(1, NUM_HIDDEN)
    gb = gate_b.reshape(1, NUM_HIDDEN)
    cols = jnp.arange(NUM_HIDDEN, dtype=jnp.int32) // HEAD_DIM
    R = (cols[None, :] == jnp.arange(NUM_HEADS, dtype=jnp.int32)[:, None]
         ).astype(f32)
    z128 = jnp.zeros((RPT, NUM_HIDDEN), f32)

    pa, pc = _node_proj(h_Vp, b1a, b1c)
    ga, gc = _sc_gather(src, dst, pa, pc)
    ex, u = _edge_stage(ga, gc, h_Ep, b1e, b1b, B2_w, b2b, b3p, b3bp,
                        W_V_w, wvb, R)
    hvn = _sc_scatter_u(src, u, z128)
    ssn = _sc_scatter_u(src, ex, z128)
    return _node_out(hvn, ssn, h_V, W_O_w, gate_w, gb)
